# Initial kernel scaffold; baseline (speedup 1.0000x reference)
import functools

import jax
import jax.numpy as jnp
from jax import lax
from jax.experimental import pallas as pl
from jax.experimental.pallas import tpu as pltpu
from jax.experimental.pallas import tpu_sc as plsc

_N = 10000
_E = 320000
_D_IN = 128
_D_H = 128
_D_CLS = 64

_NC = 2
_NS = 16
_K = 80
_EPC = _E // _NC
_EPT = _EPC // _NS
_ITERS = _EPT // _K
_RPT = _N // _NS

_CW = 16

_mesh = plsc.VectorSubcoreMesh(core_axis_name="c", subcore_axis_name="s")


def _fill_vmem(ref, rows, width, value):
    per_row = width // 16

    def body(i, _):
        r = i // per_row
        col = (i % per_row) * 16
        ref[r, pl.ds(col, 16)] = jnp.full((16,), value, jnp.float32)
        return 0

    lax.fori_loop(0, rows * per_row, body, 0)


def _zero_spmem_rows(zbuf, acc, base, total):
    done = 0
    while done < total:
        n = min(_K, total - done)
        pltpu.sync_copy(zbuf.at[pl.ds(0, n)], acc.at[pl.ds(base + done, n)])
        done += n


def _sc_degrees_body(src_hbm, dst_hbm, out_hbm, idx_s, idx_d, ones, acc_s, acc_d):
    c = lax.axis_index("c")
    s = lax.axis_index("s")

    _fill_vmem(ones, _K, _CW, 0.0)
    _zero_spmem_rows(ones, acc_s, s * _RPT, _RPT)
    _zero_spmem_rows(ones, acc_d, s * _RPT, _RPT)
    _fill_vmem(ones, _K, _CW, 1.0)
    plsc.subcore_barrier()

    def step(j, _):
        eb = c * _EPC + s * _EPT + j * _K
        pltpu.sync_copy(src_hbm.at[pl.ds(eb, _K)], idx_s)
        pltpu.sync_copy(dst_hbm.at[pl.ds(eb, _K)], idx_d)
        pltpu.sync_copy(ones, acc_s.at[idx_s], add=True)
        pltpu.sync_copy(ones, acc_d.at[idx_d], add=True)
        return 0

    lax.fori_loop(0, _ITERS, step, 0)
    plsc.subcore_barrier()

    rb = s * _RPT
    pltpu.sync_copy(acc_s.at[pl.ds(rb, _RPT)], out_hbm.at[c, 0, pl.ds(rb, _RPT)])
    pltpu.sync_copy(acc_d.at[pl.ds(rb, _RPT)], out_hbm.at[c, 1, pl.ds(rb, _RPT)])


_sc_degrees = pl.kernel(
    _sc_degrees_body,
    out_type=jax.ShapeDtypeStruct((_NC, 2, _N, _CW), jnp.float32),
    mesh=_mesh,
    scratch_types=[
        pltpu.VMEM((_K,), jnp.int32),
        pltpu.VMEM((_K,), jnp.int32),
        pltpu.VMEM((_K, _CW), jnp.float32),
        pltpu.VMEM_SHARED((_N, _CW), jnp.float32),
        pltpu.VMEM_SHARED((_N, _CW), jnp.float32),
    ],
)


def _sc_agg_body(t_hbm, src_hbm, dst_hbm, out_hbm, idx_s, idx_d, rows, acc, gsem, *, d):
    c = lax.axis_index("c")
    s = lax.axis_index("s")

    _fill_vmem(rows, _K, d, 0.0)
    _zero_spmem_rows(rows, acc, s * _RPT, _RPT)
    plsc.subcore_barrier()

    def step(j, _):
        eb = c * _EPC + s * _EPT + j * _K
        pltpu.sync_copy(src_hbm.at[pl.ds(eb, _K)], idx_s)
        pltpu.sync_copy(dst_hbm.at[pl.ds(eb, _K)], idx_d)
        pltpu.async_copy(t_hbm.at[idx_s], rows, gsem).wait()
        pltpu.sync_copy(rows, acc.at[idx_d], add=True)
        return 0

    lax.fori_loop(0, _ITERS, step, 0)
    plsc.subcore_barrier()

    rb = s * _RPT
    pltpu.sync_copy(acc.at[pl.ds(rb, _RPT)], out_hbm.at[c, pl.ds(rb, _RPT)])


def _make_sc_agg(d):
    return pl.kernel(
        functools.partial(_sc_agg_body, d=d),
        out_type=jax.ShapeDtypeStruct((_NC, _N, d), jnp.float32),
        mesh=_mesh,
        scratch_types=[
            pltpu.VMEM((_K,), jnp.int32),
            pltpu.VMEM((_K,), jnp.int32),
            pltpu.VMEM((_K, d), jnp.float32),
            pltpu.VMEM_SHARED((_N, d), jnp.float32),
            pltpu.SemaphoreType.DMA,
        ],
    )


_sc_agg128 = _make_sc_agg(_D_H)
_sc_agg64 = _make_sc_agg(_D_CLS)


def _scale_cols(cnt_ref, kind):
    c = cnt_ref[0, kind] + cnt_ref[1, kind]
    return lax.rsqrt(jnp.maximum(c[:, 0:1], 1.0))


def _tc_first_body(x_ref, w_ref, cnt_ref, o_ref):
    so = _scale_cols(cnt_ref, 0)
    o_ref[...] = (
        jnp.dot(x_ref[...], w_ref[...], preferred_element_type=jnp.float32) * so
    )


def _tc_mid_body(p_ref, cnt_ref, b_ref, w_ref, o_ref):
    si = _scale_cols(cnt_ref, 1)
    so = _scale_cols(cnt_ref, 0)
    h = jnp.maximum((p_ref[0] + p_ref[1]) * si + b_ref[...], 0.0)
    o_ref[...] = (
        jnp.dot(h, w_ref[...], preferred_element_type=jnp.float32) * so
    )


def _tc_final_body(p_ref, cnt_ref, b_ref, o_ref):
    si = _scale_cols(cnt_ref, 1)
    logits = (p_ref[0] + p_ref[1]) * si + b_ref[...]
    m = jnp.max(logits, axis=-1, keepdims=True)
    lg = logits - m
    o_ref[...] = lg - jnp.log(jnp.sum(jnp.exp(lg), axis=-1, keepdims=True))


def _tc_first(x, w, cnt):
    return pl.pallas_call(
        _tc_first_body,
        out_shape=jax.ShapeDtypeStruct((_N, w.shape[1]), jnp.float32),
    )(x, w, cnt)


def _tc_mid(p, cnt, b, w):
    return pl.pallas_call(
        _tc_mid_body,
        out_shape=jax.ShapeDtypeStruct((_N, w.shape[1]), jnp.float32),
    )(p, cnt, b, w)


def _tc_final(p, cnt, b):
    return pl.pallas_call(
        _tc_final_body,
        out_shape=jax.ShapeDtypeStruct((_N, _D_CLS), jnp.float32),
    )(p, cnt, b)


def kernel(x, edge_index, W1, b1, W2, b2, W3, b3):
    src = edge_index[0].astype(jnp.int32)
    dst = edge_index[1].astype(jnp.int32)
    x = x.astype(jnp.float32)
    b1r = b1.reshape(1, _D_H)
    b2r = b2.reshape(1, _D_H)
    b3r = b3.reshape(1, _D_CLS)

    cnt = _sc_degrees(src, dst)
    t1 = _tc_first(x, W1, cnt)
    p1 = _sc_agg128(t1, src, dst)
    t2 = _tc_mid(p1, cnt, b1r, W2)
    p2 = _sc_agg128(t2, src, dst)
    t3 = _tc_mid(p2, cnt, b2r, W3)
    p3 = _sc_agg64(t3, src, dst)
    return _tc_final(p3, cnt, b3r)


# speedup vs baseline: 4.3871x; 4.3871x over previous
import functools

import jax
import jax.numpy as jnp
from jax import lax
from jax.experimental import pallas as pl
from jax.experimental.pallas import tpu as pltpu
from jax.experimental.pallas import tpu_sc as plsc

_N = 10000
_E = 320000
_D_IN = 128
_D_H = 128
_D_CLS = 64

_NC = 2
_NS = 16
_K = 80
_EPC = _E // _NC
_EPT = _EPC // _NS
_ITERS = _EPT // _K
_NCHUNK = _N // _K

_CW = 16

_mesh = plsc.VectorSubcoreMesh(core_axis_name="c", subcore_axis_name="s")


def _fill_vmem(ref, rows, width, value):
    per_row = width // 16

    def body(i, _):
        r = i // per_row
        col = (i % per_row) * 16
        ref[r, pl.ds(col, 16)] = jnp.full((16,), value, jnp.float32)
        return 0

    lax.fori_loop(0, rows * per_row, body, 0)


def _for_my_chunks(s, fn):
    def body(i, _):
        t = i * _NS + s

        @pl.when(t < _NCHUNK)
        def _():
            fn(t * _K)

        return 0

    lax.fori_loop(0, (_NCHUNK + _NS - 1) // _NS, body, 0)


def _sc_degrees_body(pays_hbm, payd_hbm, src_hbm, dst_hbm, out_hbm,
                     idx_s, idx_d, pay_s, pay_d, acc):
    c = lax.axis_index("c")
    s = lax.axis_index("s")

    _fill_vmem(pay_s, _K, _D_H, 0.0)
    _for_my_chunks(s, lambda base: pltpu.sync_copy(pay_s, acc.at[pl.ds(base, _K)]))

    pltpu.sync_copy(pays_hbm, pay_s)
    pltpu.sync_copy(payd_hbm, pay_d)
    plsc.subcore_barrier()

    def step(j, _):
        eb = c * _EPC + s * _EPT + j * _K
        pltpu.sync_copy(src_hbm.at[pl.ds(eb, _K)], idx_s)
        pltpu.sync_copy(dst_hbm.at[pl.ds(eb, _K)], idx_d)
        pltpu.sync_copy(pay_s, acc.at[idx_s], add=True)
        pltpu.sync_copy(pay_d, acc.at[idx_d], add=True)
        return 0

    lax.fori_loop(0, _ITERS, step, 0)
    plsc.subcore_barrier()

    _for_my_chunks(
        s,
        lambda base: pltpu.sync_copy(
            acc.at[pl.ds(base, _K)], out_hbm.at[c, pl.ds(base, _K)]
        ),
    )


_sc_degrees = pl.kernel(
    _sc_degrees_body,
    out_type=jax.ShapeDtypeStruct((_NC, _N, _D_H), jnp.float32),
    mesh=_mesh,
    scratch_types=[
        pltpu.VMEM((_K,), jnp.int32),
        pltpu.VMEM((_K,), jnp.int32),
        pltpu.VMEM((_K, _D_H), jnp.float32),
        pltpu.VMEM((_K, _D_H), jnp.float32),
        pltpu.VMEM_SHARED((_N, _D_H), jnp.float32),
    ],
)


def _sc_agg_body(t_hbm, src_hbm, dst_hbm, out_hbm, idx_s, idx_d, rows, acc, gsem, *, d):
    c = lax.axis_index("c")
    s = lax.axis_index("s")

    _fill_vmem(rows, _K, d, 0.0)
    _for_my_chunks(s, lambda base: pltpu.sync_copy(rows, acc.at[pl.ds(base, _K)]))
    plsc.subcore_barrier()

    def step(j, _):
        eb = c * _EPC + s * _EPT + j * _K
        pltpu.sync_copy(src_hbm.at[pl.ds(eb, _K)], idx_s)
        pltpu.sync_copy(dst_hbm.at[pl.ds(eb, _K)], idx_d)
        pltpu.async_copy(t_hbm.at[idx_s], rows, gsem).wait()
        pltpu.sync_copy(rows, acc.at[idx_d], add=True)
        return 0

    lax.fori_loop(0, _ITERS, step, 0)
    plsc.subcore_barrier()

    _for_my_chunks(
        s,
        lambda base: pltpu.sync_copy(
            acc.at[pl.ds(base, _K)], out_hbm.at[c, pl.ds(base, _K)]
        ),
    )


def _make_sc_agg(d):
    return pl.kernel(
        functools.partial(_sc_agg_body, d=d),
        out_type=jax.ShapeDtypeStruct((_NC, _N, d), jnp.float32),
        mesh=_mesh,
        scratch_types=[
            pltpu.VMEM((_K,), jnp.int32),
            pltpu.VMEM((_K,), jnp.int32),
            pltpu.VMEM((_K, d), jnp.float32),
            pltpu.VMEM_SHARED((_N, d), jnp.float32),
            pltpu.SemaphoreType.DMA,
        ],
    )


_sc_agg128 = _make_sc_agg(_D_H)


def _scale_cols(cnt_ref, kind):
    c = cnt_ref[0][:, kind : kind + 1] + cnt_ref[1][:, kind : kind + 1]
    return lax.rsqrt(jnp.maximum(c, 1.0))


def _tc_first_body(x_ref, w_ref, cnt_ref, o_ref):
    so = _scale_cols(cnt_ref, 0)
    o_ref[...] = (
        jnp.dot(x_ref[...], w_ref[...], preferred_element_type=jnp.float32) * so
    )


def _tc_mid_body(p_ref, cnt_ref, b_ref, w_ref, o_ref):
    si = _scale_cols(cnt_ref, 1)
    so = _scale_cols(cnt_ref, 0)
    h = jnp.maximum((p_ref[0] + p_ref[1]) * si + b_ref[...], 0.0)
    o_ref[...] = (
        jnp.dot(h, w_ref[...], preferred_element_type=jnp.float32) * so
    )


def _tc_final_body(p_ref, cnt_ref, b_ref, o_ref):
    si = _scale_cols(cnt_ref, 1)
    logits = (p_ref[0] + p_ref[1])[:, : _D_CLS] * si + b_ref[...]
    m = jnp.max(logits, axis=-1, keepdims=True)
    lg = logits - m
    o_ref[...] = lg - jnp.log(jnp.sum(jnp.exp(lg), axis=-1, keepdims=True))


def _tc_first(x, w, cnt):
    return pl.pallas_call(
        _tc_first_body,
        out_shape=jax.ShapeDtypeStruct((_N, w.shape[1]), jnp.float32),
    )(x, w, cnt)


def _tc_mid(p, cnt, b, w):
    return pl.pallas_call(
        _tc_mid_body,
        out_shape=jax.ShapeDtypeStruct((_N, w.shape[1]), jnp.float32),
    )(p, cnt, b, w)


def _tc_final(p, cnt, b):
    return pl.pallas_call(
        _tc_final_body,
        out_shape=jax.ShapeDtypeStruct((_N, _D_CLS), jnp.float32),
    )(p, cnt, b)


def kernel(x, edge_index, W1, b1, W2, b2, W3, b3):
    src = edge_index[0].astype(jnp.int32)
    dst = edge_index[1].astype(jnp.int32)
    x = x.astype(jnp.float32)
    b1r = b1.reshape(1, _D_H)
    b2r = b2.reshape(1, _D_H)
    b3r = b3.reshape(1, _D_CLS)
    W3p = jnp.pad(W3, ((0, 0), (0, _D_H - _D_CLS)))

    eye = jnp.eye(2, _D_H, dtype=jnp.float32)
    pays = jnp.broadcast_to(eye[0], (_K, _D_H))
    payd = jnp.broadcast_to(eye[1], (_K, _D_H))
    cnt = _sc_degrees(pays, payd, src, dst)
    t1 = _tc_first(x, W1, cnt)
    p1 = _sc_agg128(t1, src, dst)
    t2 = _tc_mid(p1, cnt, b1r, W2)
    p2 = _sc_agg128(t2, src, dst)
    t3 = _tc_mid(p2, cnt, b2r, W3p)
    p3 = _sc_agg128(t3, src, dst)
    return _tc_final(p3, cnt, b3r)


# speedup vs baseline: 10.5007x; 2.3936x over previous
import functools

import jax
import jax.numpy as jnp
from jax import lax
from jax.experimental import pallas as pl
from jax.experimental.pallas import tpu as pltpu
from jax.experimental.pallas import tpu_sc as plsc

_N = 10000
_E = 320000
_D_IN = 128
_D_H = 128
_D_CLS = 64

_NC = 2
_NS = 16
_K = 80
_EPC = _E // _NC
_EPT = _EPC // _NS
_ITERS = _EPT // _K
_NCHUNK = _N // _K

_CW = 16

_mesh = plsc.VectorSubcoreMesh(core_axis_name="c", subcore_axis_name="s")


def _fill_vmem(ref, rows, width, value):
    per_row = width // 16

    def body(i, _):
        r = i // per_row
        col = (i % per_row) * 16
        ref[r, pl.ds(col, 16)] = jnp.full((16,), value, jnp.float32)
        return 0

    lax.fori_loop(0, rows * per_row, body, 0)


def _fill_vmem1d(ref, n, value):
    def body(i, _):
        ref[pl.ds(i * 16, 16)] = jnp.full((16,), value, jnp.float32)
        return 0

    lax.fori_loop(0, n // 16, body, 0)


def _for_my_chunks(s, fn):
    def body(i, _):
        t = i * _NS + s

        @pl.when(t < _NCHUNK)
        def _():
            fn(t * _K)

        return 0

    lax.fori_loop(0, (_NCHUNK + _NS - 1) // _NS, body, 0)


_NPAD = 10240
_NB = _NPAD // (8 * 128)
_DK = 125
_DI = 80
_DB = 16


def _sc_degrees_body(src_hbm, dst_hbm, out_hbm,
                     idx_s, idx_d, ones, stage, acc_s, acc_d, sem_a, sem_b):
    c = lax.axis_index("c")
    s = lax.axis_index("s")
    w = c * _NS + s

    _fill_vmem(stage, 8, 128, 0.0)
    nz = _NPAD // 128

    def zero_chunk(i, _):
        t = i * _NS + s

        @pl.when(t < nz)
        def _():
            off = pl.multiple_of(t * 128, 128)
            pltpu.sync_copy(stage.at[0], acc_s.at[pl.ds(off, 128)])
            pltpu.sync_copy(stage.at[0], acc_d.at[pl.ds(off, 128)])

        return 0

    lax.fori_loop(0, (nz + _NS - 1) // _NS, zero_chunk, 0)
    _fill_vmem1d(ones, 128, 1.0)
    pay = ones.at[pl.ds(0, _DK)]
    plsc.subcore_barrier()

    def blk(bi, _):
        b = pl.multiple_of(bi * _DB, 8)
        pltpu.sync_copy(src_hbm.at[w, pl.ds(b, _DB)], idx_s)
        pltpu.sync_copy(dst_hbm.at[w, pl.ds(b, _DB)], idx_d)

        def step(j, _):
            pltpu.async_copy(pay, acc_s.at[idx_s.at[j]], sem_a, add=True)
            pltpu.async_copy(pay, acc_d.at[idx_d.at[j]], sem_b, add=True)

            @pl.when(j >= 2)
            def _():
                pltpu.make_async_copy(pay, acc_s.at[idx_s.at[j - 2]], sem_a).wait()
                pltpu.make_async_copy(pay, acc_d.at[idx_d.at[j - 2]], sem_b).wait()

            return 0

        lax.fori_loop(0, _DB, step, 0)
        for j in (_DB - 2, _DB - 1):
            pltpu.make_async_copy(pay, acc_s.at[idx_s.at[j]], sem_a).wait()
            pltpu.make_async_copy(pay, acc_d.at[idx_d.at[j]], sem_b).wait()
        return 0

    lax.fori_loop(0, _DI // _DB, blk, 0)
    plsc.subcore_barrier()

    def out_block(acc, sect, blk):
        for r in range(8):
            off = pl.multiple_of(blk * 1024 + r * 128, 128)
            pltpu.sync_copy(acc.at[pl.ds(off, 128)], stage.at[r])
        row0 = pl.multiple_of(sect * (8 * _NB) + blk * 8, 8)
        pltpu.sync_copy(stage, out_hbm.at[pl.ds(row0, 8)])

    def out_step(i, _):
        t = i * _NS + s

        @pl.when(t < _NB)
        def _():
            out_block(acc_s, c, t)

        @pl.when((t >= _NB) & (t < 2 * _NB))
        def _():
            out_block(acc_d, 2 + c, t - _NB)

        return 0

    lax.fori_loop(0, (2 * _NB + _NS - 1) // _NS, out_step, 0)


_sc_degrees = pl.kernel(
    _sc_degrees_body,
    out_type=jax.ShapeDtypeStruct((4 * 8 * _NB, 128), jnp.float32),
    mesh=_mesh,
    scratch_types=[
        pltpu.VMEM((_DB, _DK), jnp.int32),
        pltpu.VMEM((_DB, _DK), jnp.int32),
        pltpu.VMEM((128,), jnp.float32),
        pltpu.VMEM((8, 128), jnp.float32),
        pltpu.VMEM_SHARED((_NPAD,), jnp.float32),
        pltpu.VMEM_SHARED((_NPAD,), jnp.float32),
        pltpu.SemaphoreType.DMA,
        pltpu.SemaphoreType.DMA,
    ],
)


_AB = 16


def _sc_agg_body(t_hbm, src_hbm, dst_hbm, out_hbm,
                 idx_s, idx_d, rows0, rows1, acc, g0, g1, s0, s1, *, d):
    c = lax.axis_index("c")
    s = lax.axis_index("s")
    w = c * _NS + s

    _fill_vmem(rows0, _K, d, 0.0)
    _for_my_chunks(
        s,
        lambda base: pltpu.sync_copy(rows0.at[pl.ds(0, _K)], acc.at[pl.ds(base, _K)]),
    )
    plsc.subcore_barrier()

    def gstart(j, rows, sem):
        pltpu.async_copy(t_hbm.at[idx_s.at[j]], rows, sem)

    def gwait(j, rows, sem):
        pltpu.make_async_copy(t_hbm.at[idx_s.at[j]], rows, sem).wait()

    def sstart(j, rows, sem):
        pltpu.async_copy(rows, acc.at[idx_d.at[j]], sem, add=True)

    def swait(j, rows, sem):
        pltpu.make_async_copy(rows, acc.at[idx_d.at[j]], sem).wait()

    def blkfn(bi, _):
        b = pl.multiple_of(bi * _AB, 8)
        pltpu.sync_copy(src_hbm.at[w, pl.ds(b, _AB)], idx_s)
        pltpu.sync_copy(dst_hbm.at[w, pl.ds(b, _AB)], idx_d)
        gstart(0, rows0, g0)

        def pair(i2, _):
            a = 2 * i2
            gwait(a, rows0, g0)
            sstart(a, rows0, s0)

            @pl.when(i2 > 0)
            def _():
                swait(a - 1, rows1, s1)

            gstart(a + 1, rows1, g1)
            gwait(a + 1, rows1, g1)
            sstart(a + 1, rows1, s1)
            swait(a, rows0, s0)

            @pl.when(i2 < _AB // 2 - 1)
            def _():
                gstart(a + 2, rows0, g0)

            return 0

        lax.fori_loop(0, _AB // 2, pair, 0)
        swait(_AB - 1, rows1, s1)
        return 0

    lax.fori_loop(0, _DI // _AB, blkfn, 0)
    plsc.subcore_barrier()

    _for_my_chunks(
        s,
        lambda base: pltpu.sync_copy(
            acc.at[pl.ds(base, _K)], out_hbm.at[c, pl.ds(base, _K)]
        ),
    )


def _make_sc_agg(d):
    return pl.kernel(
        functools.partial(_sc_agg_body, d=d),
        out_type=jax.ShapeDtypeStruct((_NC, _N, d), jnp.float32),
        mesh=_mesh,
        scratch_types=[
            pltpu.VMEM((_AB, _DK), jnp.int32),
            pltpu.VMEM((_AB, _DK), jnp.int32),
            pltpu.VMEM((_DK, d), jnp.float32),
            pltpu.VMEM((_DK, d), jnp.float32),
            pltpu.VMEM_SHARED((_N, d), jnp.float32),
            pltpu.SemaphoreType.DMA,
            pltpu.SemaphoreType.DMA,
            pltpu.SemaphoreType.DMA,
            pltpu.SemaphoreType.DMA,
        ],
    )


_sc_agg128 = _make_sc_agg(_D_H)


def _scale_cols(cnt_ref, kind):
    k = 2 * kind
    c = cnt_ref[:, k : k + 1] + cnt_ref[:, k + 1 : k + 2]
    return lax.rsqrt(jnp.maximum(c, 1.0))


def _tc_first_body(x_ref, w_ref, cnt_ref, o_ref):
    so = _scale_cols(cnt_ref, 0)
    o_ref[...] = (
        jnp.dot(x_ref[...], w_ref[...], preferred_element_type=jnp.float32) * so
    )


def _tc_mid_body(p_ref, cnt_ref, b_ref, w_ref, o_ref):
    si = _scale_cols(cnt_ref, 1)
    so = _scale_cols(cnt_ref, 0)
    h = jnp.maximum((p_ref[0] + p_ref[1]) * si + b_ref[...], 0.0)
    o_ref[...] = (
        jnp.dot(h, w_ref[...], preferred_element_type=jnp.float32) * so
    )


def _tc_final_body(p_ref, cnt_ref, b_ref, o_ref):
    si = _scale_cols(cnt_ref, 1)
    logits = (p_ref[0] + p_ref[1])[:, : _D_CLS] * si + b_ref[...]
    m = jnp.max(logits, axis=-1, keepdims=True)
    lg = logits - m
    o_ref[...] = lg - jnp.log(jnp.sum(jnp.exp(lg), axis=-1, keepdims=True))


def _tc_first(x, w, cnt):
    return pl.pallas_call(
        _tc_first_body,
        out_shape=jax.ShapeDtypeStruct((_N, w.shape[1]), jnp.float32),
    )(x, w, cnt)


def _tc_mid(p, cnt, b, w):
    return pl.pallas_call(
        _tc_mid_body,
        out_shape=jax.ShapeDtypeStruct((_N, w.shape[1]), jnp.float32),
    )(p, cnt, b, w)


def _tc_final(p, cnt, b):
    return pl.pallas_call(
        _tc_final_body,
        out_shape=jax.ShapeDtypeStruct((_N, _D_CLS), jnp.float32),
    )(p, cnt, b)


def kernel(x, edge_index, W1, b1, W2, b2, W3, b3):
    nw = _NC * _NS
    src = edge_index[0].astype(jnp.int32).reshape(nw, _DI, _DK)
    dst = edge_index[1].astype(jnp.int32).reshape(nw, _DI, _DK)
    x = x.astype(jnp.float32)
    b1r = b1.reshape(1, _D_H)
    b2r = b2.reshape(1, _D_H)
    b3r = b3.reshape(1, _D_CLS)
    W3p = jnp.pad(W3, ((0, 0), (0, _D_H - _D_CLS)))

    cnt = _sc_degrees(src, dst)
    cnt = cnt.reshape(4, _NPAD)[:, :_N].T
    t1 = _tc_first(x, W1, cnt)
    p1 = _sc_agg128(t1, src, dst)
    t2 = _tc_mid(p1, cnt, b1r, W2)
    p2 = _sc_agg128(t2, src, dst)
    t3 = _tc_mid(p2, cnt, b2r, W3p)
    p3 = _sc_agg128(t3, src, dst)
    return _tc_final(p3, cnt, b3r)
